# baseline (speedup 1.0000x reference)
import functools

import jax
import jax.numpy as jnp
from jax import lax
from jax.experimental import pallas as pl
from jax.experimental.pallas import tpu as pltpu
from jax.experimental.pallas import tpu_sc as plsc

N = 10000
E = 320000
S = 128
DEPTH = 5
NR = 32
CUTOFF = 5.0

NC = 2
NS = 16
NW = NC * NS
B = 64
NFULL = 160
EPW = NFULL * B
EPAD = EPW * NW
NPAD = 10112
ROWS_PW = NPAD // NS

EB = 512
NB = 512


@functools.cache
def _sc_mesh():
    return plsc.VectorSubcoreMesh(core_axis_name="c", subcore_axis_name="s",
                                  num_cores=NC, num_subcores=NS)


def _ssp(v):
    return jnp.maximum(v, 0.0) + jnp.log1p(jnp.exp(-jnp.abs(v))) - 0.6931471805599453


def _wf_body(nl, attr_ref, f1w_ref, f1b_ref, f2w_ref, f2b_ref, *out_refs):
    a = attr_ref[...]
    step = CUTOFF / (NR - 1)
    offs = lax.broadcasted_iota(jnp.int32, (1, NR), 1).astype(jnp.float32) * step
    coeff = -0.5 / (step * step)
    d = a - offs
    rbf = jnp.exp(coeff * d * d)
    env = 0.5 * (jnp.cos(jnp.pi / CUTOFF * a) + 1.0)
    env = jnp.where(a < CUTOFF, env, 0.0)
    for i in range(nl):
        t = _ssp(jnp.dot(rbf, f1w_ref[i], preferred_element_type=jnp.float32)
                 + f1b_ref[i])
        wf = (jnp.dot(t, f2w_ref[i], preferred_element_type=jnp.float32)
              + f2b_ref[i]) * env
        out_refs[i][...] = wf


def _wf_layers(nl, attr_col, f1_w, f1_b, f2_w, f2_b):
    ne = attr_col.shape[0]
    return pl.pallas_call(
        functools.partial(_wf_body, nl),
        grid=(ne // EB,),
        in_specs=[
            pl.BlockSpec((EB, 1), lambda e: (e, 0)),
            pl.BlockSpec((nl, NR, S), lambda e: (0, 0, 0)),
            pl.BlockSpec((nl, 1, S), lambda e: (0, 0, 0)),
            pl.BlockSpec((nl, S, S), lambda e: (0, 0, 0)),
            pl.BlockSpec((nl, 1, S), lambda e: (0, 0, 0)),
        ],
        out_specs=[pl.BlockSpec((EB, S), lambda e: (e, 0))] * nl,
        out_shape=[jax.ShapeDtypeStruct((ne, S), jnp.float32)] * nl,
    )(attr_col, f1_w, f1_b, f2_w, f2_b)


def _update_body(pa_ref, pb_ref, deg_ref, l2w_ref, l2b_ref, wn_ref, x_ref,
                 h_ref):
    agg = (pa_ref[0] + pa_ref[1]) + (pb_ref[0] + pb_ref[1])
    deg = deg_ref[0, :, 0:1] + deg_ref[1, :, 0:1]
    agg = agg / jnp.maximum(deg, 1.0)
    xn = _ssp(jnp.dot(agg, l2w_ref[...], preferred_element_type=jnp.float32)
              + l2b_ref[...])
    x_ref[...] = xn
    h_ref[...] = jnp.dot(xn, wn_ref[...], preferred_element_type=jnp.float32)


def _update(parts_a, parts_b, deg16, l2w, l2b, wnext):
    grid = (pl.cdiv(N, NB),)
    return pl.pallas_call(
        _update_body,
        grid=grid,
        in_specs=[
            pl.BlockSpec((NC, NB, S), lambda n: (0, n, 0)),
            pl.BlockSpec((NC, NB, S), lambda n: (0, n, 0)),
            pl.BlockSpec((NC, NB, S), lambda n: (0, n, 0)),
            pl.BlockSpec((S, S), lambda n: (0, 0)),
            pl.BlockSpec((1, S), lambda n: (0, 0)),
            pl.BlockSpec((S, S), lambda n: (0, 0)),
        ],
        out_specs=[
            pl.BlockSpec((NB, S), lambda n: (n, 0)),
            pl.BlockSpec((NB, S), lambda n: (n, 0)),
        ],
        out_shape=[
            jax.ShapeDtypeStruct((N, S), jnp.float32),
            jax.ShapeDtypeStruct((N, S), jnp.float32),
        ],
    )(parts_a, parts_b, deg16, l2w, l2b, wnext)


def _h0_body(x_ref, w_ref, h_ref):
    h_ref[...] = jnp.dot(x_ref[...], w_ref[...],
                         preferred_element_type=jnp.float32)


def _h0(x, w):
    return pl.pallas_call(
        _h0_body,
        grid=(pl.cdiv(N, NB),),
        in_specs=[
            pl.BlockSpec((NB, S), lambda n: (n, 0)),
            pl.BlockSpec((S, S), lambda n: (0, 0)),
        ],
        out_specs=pl.BlockSpec((NB, S), lambda n: (n, 0)),
        out_shape=jax.ShapeDtypeStruct((N, S), jnp.float32),
    )(x, w)


def _sc_agg_body(h_hbm, wf_hbm, src3_hbm, dst3_hbm, zero_hbm, out_hbm,
                 src_i, dst_i, rows0, rows1, wfv0, wfv1, outv0, outv1, agg_sh,
                 sg0, sg1, sw0, sw1, ss0, ss1, si0, si1, sd0, sd1):
    rows = (rows0, rows1)
    wfv = (wfv0, wfv1)
    outv = (outv0, outv1)
    sg = (sg0, sg1)
    sw = (sw0, sw1)
    ss = (ss0, ss1)
    si = (si0, si1)
    sd = (sd0, sd1)
    cid = lax.axis_index("c")
    sid = lax.axis_index("s")
    wid = sid * NC + cid
    rbase = sid * ROWS_PW
    pltpu.sync_copy(zero_hbm.at[pl.ds(rbase, ROWS_PW)],
                    agg_sh.at[pl.ds(rbase, ROWS_PW)])
    plsc.subcore_barrier()
    nfull = src3_hbm.shape[1]
    ebase = wid * (nfull * B)

    for k in range(2):
        pltpu.async_copy(src3_hbm.at[wid, k], src_i.at[k], si[k])
        pltpu.async_copy(dst3_hbm.at[wid, k], dst_i.at[k], sd[k])
    pltpu.make_async_copy(src3_hbm.at[wid, 0], src_i.at[0], si[0]).wait()
    pltpu.async_copy(h_hbm.at[src_i.at[0]], rows[0], sg[0])
    pltpu.async_copy(wf_hbm.at[pl.ds(ebase, B)], wfv[0], sw[0])

    @pl.loop(0, nfull, step=4)
    def _quad(c0):
        for j in range(4):
            c = c0 + j
            p = j % 2
            q = p ^ 1
            rnext = (j + 2) % 4
            rplus1 = (j + 1) % 4

            pltpu.make_async_copy(h_hbm.at[src_i.at[j]], rows[p],
                                  sg[p]).wait()
            pltpu.make_async_copy(wf_hbm.at[pl.ds(ebase, B)], wfv[p],
                                  sw[p]).wait()

            @pl.when(c + 1 < nfull)
            def _g1():
                pltpu.make_async_copy(src3_hbm.at[wid, c + 1],
                                      src_i.at[rplus1], si[q]).wait()
                pltpu.async_copy(h_hbm.at[src_i.at[rplus1]], rows[q], sg[q])
                pltpu.async_copy(
                    wf_hbm.at[pl.ds(ebase + (c + 1) * B, B)], wfv[q], sw[q])

            @pl.when(c >= 2)
            def _wdrain():
                pltpu.make_async_copy(outv[p], agg_sh.at[dst_i.at[j]],
                                      ss[p]).wait()

            @pl.loop(0, B)
            def _row(r):
                for j0 in range(0, S, 16):
                    sl = pl.ds(j0, 16)
                    outv[p][r, sl] = rows[p][r, sl] * wfv[p][r, sl]

            @pl.when(c + 2 < nfull)
            def _pf():
                pltpu.async_copy(src3_hbm.at[wid, c + 2], src_i.at[rnext],
                                 si[p])
                pltpu.async_copy(dst3_hbm.at[wid, c + 2], dst_i.at[rnext],
                                 sd[p])

            pltpu.make_async_copy(dst3_hbm.at[wid, c], dst_i.at[j],
                                  sd[p]).wait()
            pltpu.async_copy(outv[p], agg_sh.at[dst_i.at[j]], ss[p],
                             add=True)

    for p in range(2):
        pltpu.make_async_copy(outv[p], agg_sh.at[dst_i.at[0]], ss[p]).wait()
    plsc.subcore_barrier()
    pltpu.sync_copy(agg_sh.at[pl.ds(rbase, ROWS_PW)],
                    out_hbm.at[cid, pl.ds(rbase, ROWS_PW)])


@functools.cache
def _sc_agg_kernel(nfull):
    return pl.kernel(
        _sc_agg_body,
        out_type=jax.ShapeDtypeStruct((NC, NPAD, S), jnp.float32),
        mesh=_sc_mesh(),
        scratch_types=[
            pltpu.VMEM((4, B), jnp.int32),
            pltpu.VMEM((4, B), jnp.int32),
            pltpu.VMEM((B, S), jnp.float32),
            pltpu.VMEM((B, S), jnp.float32),
            pltpu.VMEM((B, S), jnp.float32),
            pltpu.VMEM((B, S), jnp.float32),
            pltpu.VMEM((B, S), jnp.float32),
            pltpu.VMEM((B, S), jnp.float32),
            pltpu.VMEM_SHARED((NPAD, S), jnp.float32),
        ] + [pltpu.SemaphoreType.DMA] * 10,
    )


def _sc_agg(h, wf, src3, dst3, zero_ns):
    return _sc_agg_kernel(src3.shape[1])(h, wf, src3, dst3, zero_ns)


def _sc_deg_body(dst3_hbm, ones_hbm, zero_hbm, out_hbm,
                 dstA, ones_v, deg_sh, sem_s):
    cid = lax.axis_index("c")
    sid = lax.axis_index("s")
    wid = sid * NC + cid
    rbase = sid * ROWS_PW
    pltpu.sync_copy(zero_hbm.at[pl.ds(rbase, ROWS_PW)],
                    deg_sh.at[pl.ds(rbase, ROWS_PW)])
    pltpu.sync_copy(dst3_hbm.at[wid], dstA)
    pltpu.sync_copy(ones_hbm, ones_v)
    plsc.subcore_barrier()

    @pl.loop(0, NFULL, step=8)
    def _wave(c0):
        for q in range(8):
            pltpu.async_copy(ones_v, deg_sh.at[dstA.at[c0 + q]], sem_s,
                             add=True)
        for q in range(8):
            pltpu.make_async_copy(ones_v, deg_sh.at[dstA.at[c0]],
                                  sem_s).wait()

    plsc.subcore_barrier()
    pltpu.sync_copy(deg_sh.at[pl.ds(rbase, ROWS_PW)],
                    out_hbm.at[cid, pl.ds(rbase, ROWS_PW)])


def _sc_deg(dst3, ones_b, zero_ns):
    k = pl.kernel(
        _sc_deg_body,
        out_type=jax.ShapeDtypeStruct((NC, NPAD, S), jnp.float32),
        mesh=_sc_mesh(),
        scratch_types=[
            pltpu.VMEM((NFULL, B), jnp.int32),
            pltpu.VMEM((B, S), jnp.float32),
            pltpu.VMEM_SHARED((NPAD, S), jnp.float32),
            pltpu.SemaphoreType.DMA,
        ],
    )
    return k(dst3, ones_b, zero_ns)


def kernel(x, edge_index, edge_attr, batch, f1_w, f1_b, f2_w, f2_b,
           lin1_w, lin2_w, lin2_b):
    del batch
    src = edge_index[0]
    dst = edge_index[1]
    pad = EPAD - E
    srcp = jnp.pad(src, (0, pad))
    dstp = jnp.pad(dst, (0, pad), constant_values=N)
    attrp = jnp.pad(edge_attr, (0, pad), constant_values=CUTOFF)
    attr_col = attrp.reshape(EPAD, 1)
    src3 = srcp.reshape(NW, NFULL, B)
    dst3 = dstp.reshape(NW, NFULL, B)
    zero_ns = jnp.zeros((NPAD, S), jnp.float32)

    f1b = f1_b.reshape(DEPTH, 1, S)
    f2b = f2_b.reshape(DEPTH, 1, S)
    half = EPAD // 2
    src3h = [srcp[:half].reshape(NW, -1, B), srcp[half:].reshape(NW, -1, B)]
    dst3h = [dstp[:half].reshape(NW, -1, B), dstp[half:].reshape(NW, -1, B)]
    wfsh = [[], []]
    for k in range(2):
        acol = attr_col[k * half:(k + 1) * half]
        (w0,) = _wf_layers(1, acol, f1_w[:1], f1b[:1], f2_w[:1], f2b[:1])
        w14 = _wf_layers(DEPTH - 1, acol, f1_w[1:], f1b[1:],
                         f2_w[1:], f2b[1:])
        wfsh[k] = [w0] + list(w14)
    ones_b = jnp.ones((B, S), jnp.float32)
    deg16 = _sc_deg(dst3, ones_b, zero_ns)
    h = _h0(x, lin1_w[0])
    fence = (wfsh[0][DEPTH - 1][0, 0] + wfsh[1][DEPTH - 1][0, 0]
             + wfsh[0][1][0, 0] + wfsh[1][1][0, 0]) * 0.0
    zdep = zero_ns + fence
    out = x
    for i in range(DEPTH):
        parts_a = _sc_agg(h, wfsh[0][i], src3h[0], dst3h[0], zdep)
        parts_b = _sc_agg(h, wfsh[1][i], src3h[1], dst3h[1], zdep)
        wnext = lin1_w[i + 1] if i + 1 < DEPTH else lin1_w[0]
        out, h = _update(parts_a, parts_b, deg16, lin2_w[i],
                         lin2_b[i].reshape(1, S), wnext)
    return out


# speedup vs baseline: 1.0822x; 1.0822x over previous
import functools

import jax
import jax.numpy as jnp
from jax import lax
from jax.experimental import pallas as pl
from jax.experimental.pallas import tpu as pltpu
from jax.experimental.pallas import tpu_sc as plsc

N = 10000
E = 320000
S = 128
DEPTH = 5
NR = 32
CUTOFF = 5.0

NC = 2
NS = 16
NW = NC * NS
B = 64
NFULL = 160
EPW = NFULL * B
EPAD = EPW * NW
NPAD = 10112
ROWS_PW = NPAD // NS

EB = 1024
NB = 512


@functools.cache
def _sc_mesh():
    return plsc.VectorSubcoreMesh(core_axis_name="c", subcore_axis_name="s",
                                  num_cores=NC, num_subcores=NS)


def _ssp(v):
    return jnp.maximum(v, 0.0) + jnp.log1p(jnp.exp(-jnp.abs(v))) - 0.6931471805599453


def _wf_body(nl, attr_ref, f1wb_ref, f2wb_ref, *out_refs):
    step = CUTOFF / (NR - 1)
    coeff = -0.5 / (step * step)
    for r in range(EB // 128):
        a = attr_ref[r:r + 1, :]
        offs = lax.broadcasted_iota(
            jnp.int32, (NR + 1, 1), 0).astype(jnp.float32) * step
        d = a - offs
        rbf = jnp.exp(coeff * d * d)
        ksub = lax.broadcasted_iota(jnp.int32, (NR + 1, 1), 0)
        rbf = jnp.where(ksub == NR, 1.0, rbf)
        env = 0.5 * (jnp.cos(jnp.pi / CUTOFF * a) + 1.0)
        env = jnp.where(a < CUTOFF, env, 0.0)
        for i in range(nl):
            t1 = lax.dot_general(
                f1wb_ref[i], rbf, (((0,), (0,)), ((), ())),
                preferred_element_type=jnp.float32)
            g = _ssp(t1) * env
            gp = jnp.concatenate([g, env], axis=0)
            wf = lax.dot_general(
                gp, f2wb_ref[i], (((0,), (0,)), ((), ())),
                preferred_element_type=jnp.float32)
            out_refs[i][pl.ds(r * 128, 128), :] = wf


def _wf_layers(nl, attr2, f1wb, f2wb):
    ne = attr2.shape[0] * 128
    return pl.pallas_call(
        functools.partial(_wf_body, nl),
        grid=(ne // EB,),
        in_specs=[
            pl.BlockSpec((EB // 128, 128), lambda e: (e, 0)),
            pl.BlockSpec((nl, NR + 1, S), lambda e: (0, 0, 0)),
            pl.BlockSpec((nl, S + 1, S), lambda e: (0, 0, 0)),
        ],
        out_specs=[pl.BlockSpec((EB, S), lambda e: (e, 0))] * nl,
        out_shape=[jax.ShapeDtypeStruct((ne, S), jnp.float32)] * nl,
    )(attr2, f1wb, f2wb)


def _update_body(pa_ref, pb_ref, deg_ref, l2w_ref, l2b_ref, wn_ref, x_ref,
                 h_ref):
    agg = (pa_ref[0] + pa_ref[1]) + (pb_ref[0] + pb_ref[1])
    deg = deg_ref[0, :, 0:1] + deg_ref[1, :, 0:1]
    agg = agg / jnp.maximum(deg, 1.0)
    xn = _ssp(jnp.dot(agg, l2w_ref[...], preferred_element_type=jnp.float32)
              + l2b_ref[...])
    x_ref[...] = xn
    h_ref[...] = jnp.dot(xn, wn_ref[...], preferred_element_type=jnp.float32)


def _update(parts_a, parts_b, deg16, l2w, l2b, wnext):
    grid = (pl.cdiv(N, NB),)
    return pl.pallas_call(
        _update_body,
        grid=grid,
        in_specs=[
            pl.BlockSpec((NC, NB, S), lambda n: (0, n, 0)),
            pl.BlockSpec((NC, NB, S), lambda n: (0, n, 0)),
            pl.BlockSpec((NC, NB, S), lambda n: (0, n, 0)),
            pl.BlockSpec((S, S), lambda n: (0, 0)),
            pl.BlockSpec((1, S), lambda n: (0, 0)),
            pl.BlockSpec((S, S), lambda n: (0, 0)),
        ],
        out_specs=[
            pl.BlockSpec((NB, S), lambda n: (n, 0)),
            pl.BlockSpec((NB, S), lambda n: (n, 0)),
        ],
        out_shape=[
            jax.ShapeDtypeStruct((N, S), jnp.float32),
            jax.ShapeDtypeStruct((N, S), jnp.float32),
        ],
    )(parts_a, parts_b, deg16, l2w, l2b, wnext)


def _h0_body(x_ref, w_ref, h_ref):
    h_ref[...] = jnp.dot(x_ref[...], w_ref[...],
                         preferred_element_type=jnp.float32)


def _h0(x, w):
    return pl.pallas_call(
        _h0_body,
        grid=(pl.cdiv(N, NB),),
        in_specs=[
            pl.BlockSpec((NB, S), lambda n: (n, 0)),
            pl.BlockSpec((S, S), lambda n: (0, 0)),
        ],
        out_specs=pl.BlockSpec((NB, S), lambda n: (n, 0)),
        out_shape=jax.ShapeDtypeStruct((N, S), jnp.float32),
    )(x, w)


def _sc_agg_body(h_hbm, wf_hbm, src3_hbm, dst3_hbm, zero_hbm, out_hbm,
                 src_i, dst_i, rows0, rows1, wfv0, wfv1, outv0, outv1, agg_sh,
                 sg0, sg1, sw0, sw1, ss0, ss1, si0, si1, sd0, sd1):
    rows = (rows0, rows1)
    wfv = (wfv0, wfv1)
    outv = (outv0, outv1)
    sg = (sg0, sg1)
    sw = (sw0, sw1)
    ss = (ss0, ss1)
    si = (si0, si1)
    sd = (sd0, sd1)
    cid = lax.axis_index("c")
    sid = lax.axis_index("s")
    wid = sid * NC + cid
    rbase = sid * ROWS_PW
    pltpu.sync_copy(zero_hbm.at[pl.ds(rbase, ROWS_PW)],
                    agg_sh.at[pl.ds(rbase, ROWS_PW)])
    plsc.subcore_barrier()
    nfull = src3_hbm.shape[1]
    ebase = wid * (nfull * B)

    for k in range(2):
        pltpu.async_copy(src3_hbm.at[wid, k], src_i.at[k], si[k])
        pltpu.async_copy(dst3_hbm.at[wid, k], dst_i.at[k], sd[k])
    pltpu.make_async_copy(src3_hbm.at[wid, 0], src_i.at[0], si[0]).wait()
    pltpu.async_copy(h_hbm.at[src_i.at[0]], rows[0], sg[0])
    pltpu.async_copy(wf_hbm.at[pl.ds(ebase, B)], wfv[0], sw[0])

    @pl.loop(0, nfull, step=4)
    def _quad(c0):
        for j in range(4):
            c = c0 + j
            p = j % 2
            q = p ^ 1
            rnext = (j + 2) % 4
            rplus1 = (j + 1) % 4

            pltpu.make_async_copy(h_hbm.at[src_i.at[j]], rows[p],
                                  sg[p]).wait()
            pltpu.make_async_copy(wf_hbm.at[pl.ds(ebase, B)], wfv[p],
                                  sw[p]).wait()

            @pl.when(c + 1 < nfull)
            def _g1():
                pltpu.make_async_copy(src3_hbm.at[wid, c + 1],
                                      src_i.at[rplus1], si[q]).wait()
                pltpu.async_copy(h_hbm.at[src_i.at[rplus1]], rows[q], sg[q])
                pltpu.async_copy(
                    wf_hbm.at[pl.ds(ebase + (c + 1) * B, B)], wfv[q], sw[q])

            @pl.when(c >= 2)
            def _wdrain():
                pltpu.make_async_copy(outv[p], agg_sh.at[dst_i.at[j]],
                                      ss[p]).wait()

            @pl.loop(0, B)
            def _row(r):
                for j0 in range(0, S, 16):
                    sl = pl.ds(j0, 16)
                    outv[p][r, sl] = rows[p][r, sl] * wfv[p][r, sl]

            @pl.when(c + 2 < nfull)
            def _pf():
                pltpu.async_copy(src3_hbm.at[wid, c + 2], src_i.at[rnext],
                                 si[p])
                pltpu.async_copy(dst3_hbm.at[wid, c + 2], dst_i.at[rnext],
                                 sd[p])

            pltpu.make_async_copy(dst3_hbm.at[wid, c], dst_i.at[j],
                                  sd[p]).wait()
            pltpu.async_copy(outv[p], agg_sh.at[dst_i.at[j]], ss[p],
                             add=True)

    for p in range(2):
        pltpu.make_async_copy(outv[p], agg_sh.at[dst_i.at[0]], ss[p]).wait()
    plsc.subcore_barrier()
    pltpu.sync_copy(agg_sh.at[pl.ds(rbase, ROWS_PW)],
                    out_hbm.at[cid, pl.ds(rbase, ROWS_PW)])


@functools.cache
def _sc_agg_kernel(nfull):
    return pl.kernel(
        _sc_agg_body,
        out_type=jax.ShapeDtypeStruct((NC, NPAD, S), jnp.float32),
        mesh=_sc_mesh(),
        scratch_types=[
            pltpu.VMEM((4, B), jnp.int32),
            pltpu.VMEM((4, B), jnp.int32),
            pltpu.VMEM((B, S), jnp.float32),
            pltpu.VMEM((B, S), jnp.float32),
            pltpu.VMEM((B, S), jnp.float32),
            pltpu.VMEM((B, S), jnp.float32),
            pltpu.VMEM((B, S), jnp.float32),
            pltpu.VMEM((B, S), jnp.float32),
            pltpu.VMEM_SHARED((NPAD, S), jnp.float32),
        ] + [pltpu.SemaphoreType.DMA] * 10,
    )


def _sc_agg(h, wf, src3, dst3, zero_ns):
    return _sc_agg_kernel(src3.shape[1])(h, wf, src3, dst3, zero_ns)


def _sc_deg_body(dst3_hbm, ones_hbm, zero_hbm, out_hbm,
                 dstA, ones_v, deg_sh, sem_s):
    cid = lax.axis_index("c")
    sid = lax.axis_index("s")
    wid = sid * NC + cid
    rbase = sid * ROWS_PW
    pltpu.sync_copy(zero_hbm.at[pl.ds(rbase, ROWS_PW)],
                    deg_sh.at[pl.ds(rbase, ROWS_PW)])
    pltpu.sync_copy(dst3_hbm.at[wid], dstA)
    pltpu.sync_copy(ones_hbm, ones_v)
    plsc.subcore_barrier()

    @pl.loop(0, NFULL, step=8)
    def _wave(c0):
        for q in range(8):
            pltpu.async_copy(ones_v, deg_sh.at[dstA.at[c0 + q]], sem_s,
                             add=True)
        for q in range(8):
            pltpu.make_async_copy(ones_v, deg_sh.at[dstA.at[c0]],
                                  sem_s).wait()

    plsc.subcore_barrier()
    pltpu.sync_copy(deg_sh.at[pl.ds(rbase, ROWS_PW)],
                    out_hbm.at[cid, pl.ds(rbase, ROWS_PW)])


def _sc_deg(dst3, ones_b, zero_ns):
    k = pl.kernel(
        _sc_deg_body,
        out_type=jax.ShapeDtypeStruct((NC, NPAD, S), jnp.float32),
        mesh=_sc_mesh(),
        scratch_types=[
            pltpu.VMEM((NFULL, B), jnp.int32),
            pltpu.VMEM((B, S), jnp.float32),
            pltpu.VMEM_SHARED((NPAD, S), jnp.float32),
            pltpu.SemaphoreType.DMA,
        ],
    )
    return k(dst3, ones_b, zero_ns)


def kernel(x, edge_index, edge_attr, batch, f1_w, f1_b, f2_w, f2_b,
           lin1_w, lin2_w, lin2_b):
    del batch
    src = edge_index[0]
    dst = edge_index[1]
    pad = EPAD - E
    srcp = jnp.pad(src, (0, pad))
    dstp = jnp.pad(dst, (0, pad), constant_values=N)
    attrp = jnp.pad(edge_attr, (0, pad), constant_values=CUTOFF)
    src3 = srcp.reshape(NW, NFULL, B)
    dst3 = dstp.reshape(NW, NFULL, B)
    zero_ns = jnp.zeros((NPAD, S), jnp.float32)

    f1wb = jnp.concatenate([f1_w, f1_b[:, None, :]], axis=1)
    f2wb = jnp.concatenate([f2_w, f2_b[:, None, :]], axis=1)
    half = EPAD // 2
    src3h = [srcp[:half].reshape(NW, -1, B), srcp[half:].reshape(NW, -1, B)]
    dst3h = [dstp[:half].reshape(NW, -1, B), dstp[half:].reshape(NW, -1, B)]
    wfsh = [[], []]
    for k in range(2):
        a2 = attrp[k * half:(k + 1) * half].reshape(-1, 128)
        (w0,) = _wf_layers(1, a2, f1wb[:1], f2wb[:1])
        w14 = _wf_layers(DEPTH - 1, a2, f1wb[1:], f2wb[1:])
        wfsh[k] = [w0] + list(w14)
    ones_b = jnp.ones((B, S), jnp.float32)
    deg16 = _sc_deg(dst3, ones_b, zero_ns)
    h = _h0(x, lin1_w[0])
    out = x
    for i in range(DEPTH):
        parts_a = _sc_agg(h, wfsh[0][i], src3h[0], dst3h[0], zero_ns)
        parts_b = _sc_agg(h, wfsh[1][i], src3h[1], dst3h[1], zero_ns)
        wnext = lin1_w[i + 1] if i + 1 < DEPTH else lin1_w[0]
        out, h = _update(parts_a, parts_b, deg16, lin2_w[i],
                         lin2_b[i].reshape(1, S), wnext)
    return out


# speedup vs baseline: 1.4022x; 1.2956x over previous
import functools

import jax
import jax.numpy as jnp
from jax import lax
from jax.experimental import pallas as pl
from jax.experimental.pallas import tpu as pltpu
from jax.experimental.pallas import tpu_sc as plsc

N = 10000
E = 320000
S = 128
DEPTH = 5
NR = 32
CUTOFF = 5.0

NC = 2
NS = 16
NW = NC * NS
B = 64
NFULL = 160
EPW = NFULL * B
EPAD = EPW * NW
NPAD = 10112
ROWS_PW = NPAD // NS

EB = 1024
NB = 512


@functools.cache
def _sc_mesh():
    return plsc.VectorSubcoreMesh(core_axis_name="c", subcore_axis_name="s",
                                  num_cores=NC, num_subcores=NS)


def _ssp(v):
    return jnp.maximum(v, 0.0) + jnp.log1p(jnp.exp(-jnp.abs(v))) - 0.6931471805599453


def _wf_body(nl, attr_ref, f1wb_ref, f2wb_ref, *out_refs):
    step = CUTOFF / (NR - 1)
    coeff = -0.5 / (step * step)
    for r in range(EB // 128):
        a = attr_ref[r:r + 1, :]
        offs = lax.broadcasted_iota(
            jnp.int32, (NR + 1, 1), 0).astype(jnp.float32) * step
        d = a - offs
        rbf = jnp.exp(coeff * d * d)
        ksub = lax.broadcasted_iota(jnp.int32, (NR + 1, 1), 0)
        rbf = jnp.where(ksub == NR, 1.0, rbf)
        env = 0.5 * (jnp.cos(jnp.pi / CUTOFF * a) + 1.0)
        env = jnp.where(a < CUTOFF, env, 0.0)
        for i in range(nl):
            t1 = lax.dot_general(
                f1wb_ref[i], rbf, (((0,), (0,)), ((), ())),
                preferred_element_type=jnp.float32)
            g = _ssp(t1) * env
            gp = jnp.concatenate([g, env], axis=0)
            wf = lax.dot_general(
                gp, f2wb_ref[i], (((0,), (0,)), ((), ())),
                preferred_element_type=jnp.float32)
            out_refs[i][pl.ds(r * 128, 128), :] = wf


def _wf_layers(nl, attr2, f1wb, f2wb):
    ne = attr2.shape[0] * 128
    return pl.pallas_call(
        functools.partial(_wf_body, nl),
        grid=(ne // EB,),
        in_specs=[
            pl.BlockSpec((EB // 128, 128), lambda e: (e, 0)),
            pl.BlockSpec((nl, NR + 1, S), lambda e: (0, 0, 0)),
            pl.BlockSpec((nl, S + 1, S), lambda e: (0, 0, 0)),
        ],
        out_specs=[pl.BlockSpec((EB, S), lambda e: (e, 0))] * nl,
        out_shape=[jax.ShapeDtypeStruct((ne, S), jnp.float32)] * nl,
    )(attr2, f1wb, f2wb)


def _update_body(pa_ref, pb_ref, deg_ref, l2w_ref, l2b_ref, wn_ref, x_ref,
                 h_ref):
    agg = (pa_ref[0] + pa_ref[1]) + (pb_ref[0] + pb_ref[1])
    deg = deg_ref[0, :, 0:1] + deg_ref[1, :, 0:1]
    agg = agg / jnp.maximum(deg, 1.0)
    xn = _ssp(jnp.dot(agg, l2w_ref[...], preferred_element_type=jnp.float32)
              + l2b_ref[...])
    x_ref[...] = xn
    h_ref[...] = jnp.dot(xn, wn_ref[...], preferred_element_type=jnp.float32)


def _update(parts_a, parts_b, deg16, l2w, l2b, wnext):
    grid = (pl.cdiv(N, NB),)
    return pl.pallas_call(
        _update_body,
        grid=grid,
        in_specs=[
            pl.BlockSpec((NC, NB, S), lambda n: (0, n, 0)),
            pl.BlockSpec((NC, NB, S), lambda n: (0, n, 0)),
            pl.BlockSpec((NC, NB, S), lambda n: (0, n, 0)),
            pl.BlockSpec((S, S), lambda n: (0, 0)),
            pl.BlockSpec((1, S), lambda n: (0, 0)),
            pl.BlockSpec((S, S), lambda n: (0, 0)),
        ],
        out_specs=[
            pl.BlockSpec((NB, S), lambda n: (n, 0)),
            pl.BlockSpec((NB, S), lambda n: (n, 0)),
        ],
        out_shape=[
            jax.ShapeDtypeStruct((N, S), jnp.float32),
            jax.ShapeDtypeStruct((N, S), jnp.float32),
        ],
    )(parts_a, parts_b, deg16, l2w, l2b, wnext)


def _h0_body(x_ref, w_ref, h_ref):
    h_ref[...] = jnp.dot(x_ref[...], w_ref[...],
                         preferred_element_type=jnp.float32)


def _h0(x, w):
    return pl.pallas_call(
        _h0_body,
        grid=(pl.cdiv(N, NB),),
        in_specs=[
            pl.BlockSpec((NB, S), lambda n: (n, 0)),
            pl.BlockSpec((S, S), lambda n: (0, 0)),
        ],
        out_specs=pl.BlockSpec((NB, S), lambda n: (n, 0)),
        out_shape=jax.ShapeDtypeStruct((N, S), jnp.float32),
    )(x, w)


def _sc_agg_body(h_hbm, wf_hbm, src3_hbm, dst3_hbm, zero_hbm, out_hbm,
                 src_i, dst_i, rows0, rows1, wfv0, wfv1, outv0, outv1, agg_sh,
                 sg0, sg1, sw0, sw1, ss0, ss1, si0, si1, sd0, sd1):
    rows = (rows0, rows1)
    wfv = (wfv0, wfv1)
    outv = (outv0, outv1)
    sg = (sg0, sg1)
    sw = (sw0, sw1)
    ss = (ss0, ss1)
    si = (si0, si1)
    sd = (sd0, sd1)
    cid = lax.axis_index("c")
    sid = lax.axis_index("s")
    wid = sid * NC + cid
    rbase = sid * ROWS_PW
    pltpu.sync_copy(zero_hbm.at[pl.ds(rbase, ROWS_PW)],
                    agg_sh.at[pl.ds(rbase, ROWS_PW)])
    plsc.subcore_barrier()
    nfull = src3_hbm.shape[1]
    ebase = wid * (nfull * B)

    for k in range(2):
        pltpu.async_copy(src3_hbm.at[wid, k], src_i.at[k], si[k])
        pltpu.async_copy(dst3_hbm.at[wid, k], dst_i.at[k], sd[k])
    pltpu.make_async_copy(src3_hbm.at[wid, 0], src_i.at[0], si[0]).wait()
    pltpu.async_copy(h_hbm.at[src_i.at[0]], rows[0], sg[0])
    pltpu.async_copy(wf_hbm.at[pl.ds(ebase, B)], wfv[0], sw[0])

    @pl.loop(0, nfull, step=4)
    def _quad(c0):
        for j in range(4):
            c = c0 + j
            p = j % 2
            q = p ^ 1
            rnext = (j + 2) % 4
            rplus1 = (j + 1) % 4

            pltpu.make_async_copy(h_hbm.at[src_i.at[j]], rows[p],
                                  sg[p]).wait()
            pltpu.make_async_copy(wf_hbm.at[pl.ds(ebase, B)], wfv[p],
                                  sw[p]).wait()

            @pl.when(c + 1 < nfull)
            def _g1():
                pltpu.make_async_copy(src3_hbm.at[wid, c + 1],
                                      src_i.at[rplus1], si[q]).wait()
                pltpu.async_copy(h_hbm.at[src_i.at[rplus1]], rows[q], sg[q])
                pltpu.async_copy(
                    wf_hbm.at[pl.ds(ebase + (c + 1) * B, B)], wfv[q], sw[q])

            @pl.when(c >= 2)
            def _wdrain():
                pltpu.make_async_copy(outv[p], agg_sh.at[dst_i.at[j]],
                                      ss[p]).wait()

            @pl.loop(0, B)
            def _row(r):
                for j0 in range(0, S, 16):
                    sl = pl.ds(j0, 16)
                    outv[p][r, sl] = rows[p][r, sl] * wfv[p][r, sl]

            @pl.when(c + 2 < nfull)
            def _pf():
                pltpu.async_copy(src3_hbm.at[wid, c + 2], src_i.at[rnext],
                                 si[p])
                pltpu.async_copy(dst3_hbm.at[wid, c + 2], dst_i.at[rnext],
                                 sd[p])

            pltpu.make_async_copy(dst3_hbm.at[wid, c], dst_i.at[j],
                                  sd[p]).wait()
            pltpu.async_copy(outv[p], agg_sh.at[dst_i.at[j]], ss[p],
                             add=True)

    for p in range(2):
        pltpu.make_async_copy(outv[p], agg_sh.at[dst_i.at[0]], ss[p]).wait()
    plsc.subcore_barrier()
    pltpu.sync_copy(agg_sh.at[pl.ds(rbase, ROWS_PW)],
                    out_hbm.at[cid, pl.ds(rbase, ROWS_PW)])


@functools.cache
def _sc_agg_kernel(nfull):
    return pl.kernel(
        _sc_agg_body,
        out_type=jax.ShapeDtypeStruct((NC, NPAD, S), jnp.float32),
        mesh=_sc_mesh(),
        scratch_types=[
            pltpu.VMEM((4, B), jnp.int32),
            pltpu.VMEM((4, B), jnp.int32),
            pltpu.VMEM((B, S), jnp.float32),
            pltpu.VMEM((B, S), jnp.float32),
            pltpu.VMEM((B, S), jnp.float32),
            pltpu.VMEM((B, S), jnp.float32),
            pltpu.VMEM((B, S), jnp.float32),
            pltpu.VMEM((B, S), jnp.float32),
            pltpu.VMEM_SHARED((NPAD, S), jnp.float32),
        ] + [pltpu.SemaphoreType.DMA] * 10,
    )


def _sc_agg(h, wf, src3, dst3, zero_ns):
    return _sc_agg_kernel(src3.shape[1])(h, wf, src3, dst3, zero_ns)


def _sc_deg_body(dst3_hbm, ones_hbm, zero_hbm, out_hbm,
                 dstA, ones_v, deg_sh, sem_s):
    cid = lax.axis_index("c")
    sid = lax.axis_index("s")
    wid = sid * NC + cid
    rbase = sid * ROWS_PW
    pltpu.sync_copy(zero_hbm.at[pl.ds(rbase, ROWS_PW)],
                    deg_sh.at[pl.ds(rbase, ROWS_PW)])
    pltpu.sync_copy(dst3_hbm.at[wid], dstA)
    pltpu.sync_copy(ones_hbm, ones_v)
    plsc.subcore_barrier()

    @pl.loop(0, NFULL, step=8)
    def _wave(c0):
        for q in range(8):
            pltpu.async_copy(ones_v, deg_sh.at[dstA.at[c0 + q]], sem_s,
                             add=True)
        for q in range(8):
            pltpu.make_async_copy(ones_v, deg_sh.at[dstA.at[c0]],
                                  sem_s).wait()

    plsc.subcore_barrier()
    pltpu.sync_copy(deg_sh.at[pl.ds(rbase, ROWS_PW)],
                    out_hbm.at[cid, pl.ds(rbase, ROWS_PW)])


def _sc_deg(dst3, ones_b, zero_ns):
    k = pl.kernel(
        _sc_deg_body,
        out_type=jax.ShapeDtypeStruct((NC, NPAD, S), jnp.float32),
        mesh=_sc_mesh(),
        scratch_types=[
            pltpu.VMEM((NFULL, B), jnp.int32),
            pltpu.VMEM((B, S), jnp.float32),
            pltpu.VMEM_SHARED((NPAD, S), jnp.float32),
            pltpu.SemaphoreType.DMA,
        ],
    )
    return k(dst3, ones_b, zero_ns)


def kernel(x, edge_index, edge_attr, batch, f1_w, f1_b, f2_w, f2_b,
           lin1_w, lin2_w, lin2_b):
    del batch
    src = edge_index[0]
    dst = edge_index[1]
    pad = EPAD - E
    srcp = jnp.pad(src, (0, pad))
    dstp = jnp.pad(dst, (0, pad), constant_values=N)
    attrp = jnp.pad(edge_attr, (0, pad), constant_values=CUTOFF)
    src3 = srcp.reshape(NW, NFULL, B)
    dst3 = dstp.reshape(NW, NFULL, B)
    zero_ns = jnp.zeros((NPAD, S), jnp.float32)

    f1wb = jnp.concatenate([f1_w, f1_b[:, None, :]], axis=1)
    f2wb = jnp.concatenate([f2_w, f2_b[:, None, :]], axis=1)
    half = EPAD // 2
    src3h = [srcp[:half].reshape(NW, -1, B), srcp[half:].reshape(NW, -1, B)]
    dst3h = [dstp[:half].reshape(NW, -1, B), dstp[half:].reshape(NW, -1, B)]
    wfsh = [[], []]
    for k in range(2):
        a2 = attrp[k * half:(k + 1) * half].reshape(-1, 128)
        for i in range(DEPTH):
            (wl,) = _wf_layers(1, a2, f1wb[i:i + 1], f2wb[i:i + 1])
            wfsh[k].append(wl)
    ones_b = jnp.ones((B, S), jnp.float32)
    deg16 = _sc_deg(dst3, ones_b, zero_ns)
    h = _h0(x, lin1_w[0])
    out = x
    for i in range(DEPTH):
        parts_a = _sc_agg(h, wfsh[0][i], src3h[0], dst3h[0], zero_ns)
        parts_b = _sc_agg(h, wfsh[1][i], src3h[1], dst3h[1], zero_ns)
        wnext = lin1_w[i + 1] if i + 1 < DEPTH else lin1_w[0]
        out, h = _update(parts_a, parts_b, deg16, lin2_w[i],
                         lin2_b[i].reshape(1, S), wnext)
    return out
